# baseline (device time: 20033 ns/iter reference)
import jax
import jax.numpy as jnp
from jax import lax
from jax.experimental import pallas as pl
from jax.experimental.pallas import tpu as pltpu

CQ = 32
K2 = 4
K1 = 12
NF = CQ - K2
ND = CQ - K1
HD = ND // 2
NX = CQ + 2 * K2 + K1


def kernel(x):
    m, n = x.shape
    quarter = m // 4
    rows = quarter // CQ

    def body(x_ref, out_ref, send_buf, comm_x, comm_y, comm_z, comm_d,
             x_s, x_r, yf_s, yf_r, zf_s, zf_r, yr_s, yr_r, zr_s, zr_r):
        my_x = lax.axis_index("x")
        my_y = lax.axis_index("y")
        my_z = lax.axis_index("z")
        p = lax.rem(my_z, 2)
        peer_x = (1 - my_x, my_y, my_z)
        peer_y = (my_x, 1 - my_y, my_z)
        peer_z = (my_x, my_y, my_z + 1 - 2 * p)

        off_m = (2 * p + my_y) * quarter
        off_y = (2 * p + (1 - my_y)) * quarter
        off_z = (2 * (1 - p) + my_y) * quarter
        off_d = (2 * (1 - p) + (1 - my_y)) * quarter

        slots = (
            [(c, off_m, c) for c in range(CQ)]
            + [(CQ + i, off_y, NF + i) for i in range(K2)]
            + [(CQ + K2 + i, off_z, NF + i) for i in range(K2)]
            + [(CQ + 2 * K2 + i, off_d, ND + i) for i in range(K1)]
        )

        barrier = pltpu.get_barrier_semaphore()
        for nbr in (peer_x, peer_y, peer_z):
            pl.semaphore_signal(
                barrier, inc=1, device_id=nbr,
                device_id_type=pl.DeviceIdType.MESH,
            )
        pl.semaphore_wait(barrier, 3)

        x_rd = []
        for slot, off, c in slots:
            send_buf[slot, :, :] = (
                x_ref[pl.ds(off + c * rows, rows), :].astype(jnp.bfloat16)
            )
            r = pltpu.make_async_remote_copy(
                src_ref=send_buf.at[slot],
                dst_ref=comm_x.at[slot],
                send_sem=x_s.at[slot],
                recv_sem=x_r.at[slot],
                device_id=peer_x,
                device_id_type=pl.DeviceIdType.MESH,
            )
            r.start()
            x_rd.append(r)

        def fold(off, c, buf, slot):
            out_ref[pl.ds(off + c * rows, rows), :] = (
                x_ref[pl.ds(off + c * rows, rows), :]
                + buf[slot, :, :].astype(jnp.float32)
            ).astype(jnp.bfloat16)

        yf_rd, zf_rd = [], []
        for c in range(NF):
            x_rd[c].wait_recv()
            for dst, ss, rs, peer, lst in (
                (comm_y, yf_s, yf_r, peer_y, yf_rd),
                (comm_z, zf_s, zf_r, peer_z, zf_rd),
            ):
                r = pltpu.make_async_remote_copy(
                    src_ref=comm_x.at[c], dst_ref=dst.at[c],
                    send_sem=ss.at[c], recv_sem=rs.at[c],
                    device_id=peer, device_id_type=pl.DeviceIdType.MESH,
                )
                r.start()
                lst.append(r)
            out_ref[pl.ds(off_m + c * rows, rows), :] = (
                send_buf[c, :, :] + comm_x[c, :, :]
            )

        yr_rd, zr_rd = [], []
        for c in range(HD):
            zf_rd[c].wait_recv()
            r = pltpu.make_async_remote_copy(
                src_ref=comm_z.at[c], dst_ref=comm_d.at[c],
                send_sem=yr_s.at[c], recv_sem=yr_r.at[c],
                device_id=peer_y, device_id_type=pl.DeviceIdType.MESH,
            )
            r.start()
            yr_rd.append(r)
            fold(off_z, c, comm_z, c)
            yf_rd[HD + c].wait_recv()
            r = pltpu.make_async_remote_copy(
                src_ref=comm_y.at[HD + c], dst_ref=comm_d.at[HD + c],
                send_sem=zr_s.at[c], recv_sem=zr_r.at[c],
                device_id=peer_z, device_id_type=pl.DeviceIdType.MESH,
            )
            r.start()
            zr_rd.append(r)
            fold(off_y, HD + c, comm_y, HD + c)

        for c in range(HD):
            yf_rd[c].wait_recv()
            fold(off_y, c, comm_y, c)
        for c in range(2 * HD, NF):
            yf_rd[c].wait_recv()
            fold(off_y, c, comm_y, c)
        for c in range(HD, NF):
            zf_rd[c].wait_recv()
            fold(off_z, c, comm_z, c)

        for i, (slot, off, c) in enumerate(slots[CQ:]):
            x_rd[CQ + i].wait_recv()
            fold(off, c, comm_x, slot)
        for c in range(NF, CQ):
            x_rd[c].wait_recv()
            out_ref[pl.ds(off_m + c * rows, rows), :] = (
                send_buf[c, :, :] + comm_x[c, :, :]
            )

        for c in range(HD):
            yr_rd[c].wait_recv()
            fold(off_d, c, comm_d, c)
        for c in range(HD):
            zr_rd[c].wait_recv()
            fold(off_d, HD + c, comm_d, HD + c)

        for r in x_rd + yf_rd + zf_rd + yr_rd + zr_rd:
            r.wait_send()

    return pl.pallas_call(
        body,
        out_shape=jax.ShapeDtypeStruct((m, n), jnp.bfloat16),
        in_specs=[pl.BlockSpec(memory_space=pltpu.VMEM)],
        out_specs=pl.BlockSpec(memory_space=pltpu.VMEM),
        scratch_shapes=[
            pltpu.VMEM((NX, rows, n), jnp.bfloat16),
            pltpu.VMEM((NX, rows, n), jnp.bfloat16),
            pltpu.VMEM((NF, rows, n), jnp.bfloat16),
            pltpu.VMEM((NF, rows, n), jnp.bfloat16),
            pltpu.VMEM((ND, rows, n), jnp.bfloat16),
            pltpu.SemaphoreType.DMA((NX,)),
            pltpu.SemaphoreType.DMA((NX,)),
            pltpu.SemaphoreType.DMA((NF,)),
            pltpu.SemaphoreType.DMA((NF,)),
            pltpu.SemaphoreType.DMA((NF,)),
            pltpu.SemaphoreType.DMA((NF,)),
            pltpu.SemaphoreType.DMA((HD,)),
            pltpu.SemaphoreType.DMA((HD,)),
            pltpu.SemaphoreType.DMA((HD,)),
            pltpu.SemaphoreType.DMA((HD,)),
        ],
        compiler_params=pltpu.CompilerParams(collective_id=0),
    )(x)


# device time: 18834 ns/iter; 1.0637x vs baseline; 1.0637x over previous
import jax
import jax.numpy as jnp
from jax import lax
from jax.experimental import pallas as pl
from jax.experimental.pallas import tpu as pltpu

CQ = 8
K2 = 1
K1 = 2
NF = CQ - K2
ND = CQ - K1
HD = ND // 2
NX = CQ + 2 * K2 + K1


def kernel(x):
    m, n = x.shape
    quarter = m // 4
    rows = quarter // CQ

    def body(x_ref, out_ref, send_buf, comm_x, comm_y, comm_z, comm_d,
             x_s, x_r, yf_s, yf_r, zf_s, zf_r, yr_s, yr_r, zr_s, zr_r):
        my_x = lax.axis_index("x")
        my_y = lax.axis_index("y")
        my_z = lax.axis_index("z")
        p = lax.rem(my_z, 2)
        peer_x = (1 - my_x, my_y, my_z)
        peer_y = (my_x, 1 - my_y, my_z)
        peer_z = (my_x, my_y, my_z + 1 - 2 * p)

        off_m = (2 * p + my_y) * quarter
        off_y = (2 * p + (1 - my_y)) * quarter
        off_z = (2 * (1 - p) + my_y) * quarter
        off_d = (2 * (1 - p) + (1 - my_y)) * quarter

        slots = (
            [(c, off_m, c) for c in range(CQ)]
            + [(CQ + i, off_y, NF + i) for i in range(K2)]
            + [(CQ + K2 + i, off_z, NF + i) for i in range(K2)]
            + [(CQ + 2 * K2 + i, off_d, ND + i) for i in range(K1)]
        )

        barrier = pltpu.get_barrier_semaphore()
        for nbr in (peer_x, peer_y, peer_z):
            pl.semaphore_signal(
                barrier, inc=1, device_id=nbr,
                device_id_type=pl.DeviceIdType.MESH,
            )
        pl.semaphore_wait(barrier, 3)

        x_rd = []
        for slot, off, c in slots:
            send_buf[slot, :, :] = (
                x_ref[pl.ds(off + c * rows, rows), :].astype(jnp.bfloat16)
            )
            r = pltpu.make_async_remote_copy(
                src_ref=send_buf.at[slot],
                dst_ref=comm_x.at[slot],
                send_sem=x_s.at[slot],
                recv_sem=x_r.at[slot],
                device_id=peer_x,
                device_id_type=pl.DeviceIdType.MESH,
            )
            r.start()
            x_rd.append(r)

        def fold(off, c, buf, slot):
            out_ref[pl.ds(off + c * rows, rows), :] = (
                x_ref[pl.ds(off + c * rows, rows), :]
                + buf[slot, :, :].astype(jnp.float32)
            ).astype(jnp.bfloat16)

        yf_rd, zf_rd = [], []
        for c in range(NF):
            x_rd[c].wait_recv()
            for dst, ss, rs, peer, lst in (
                (comm_y, yf_s, yf_r, peer_y, yf_rd),
                (comm_z, zf_s, zf_r, peer_z, zf_rd),
            ):
                r = pltpu.make_async_remote_copy(
                    src_ref=comm_x.at[c], dst_ref=dst.at[c],
                    send_sem=ss.at[c], recv_sem=rs.at[c],
                    device_id=peer, device_id_type=pl.DeviceIdType.MESH,
                )
                r.start()
                lst.append(r)
            out_ref[pl.ds(off_m + c * rows, rows), :] = (
                send_buf[c, :, :] + comm_x[c, :, :]
            )

        yr_rd, zr_rd = [], []
        for c in range(HD):
            zf_rd[c].wait_recv()
            r = pltpu.make_async_remote_copy(
                src_ref=comm_z.at[c], dst_ref=comm_d.at[c],
                send_sem=yr_s.at[c], recv_sem=yr_r.at[c],
                device_id=peer_y, device_id_type=pl.DeviceIdType.MESH,
            )
            r.start()
            yr_rd.append(r)
            fold(off_z, c, comm_z, c)
            yf_rd[HD + c].wait_recv()
            r = pltpu.make_async_remote_copy(
                src_ref=comm_y.at[HD + c], dst_ref=comm_d.at[HD + c],
                send_sem=zr_s.at[c], recv_sem=zr_r.at[c],
                device_id=peer_z, device_id_type=pl.DeviceIdType.MESH,
            )
            r.start()
            zr_rd.append(r)
            fold(off_y, HD + c, comm_y, HD + c)

        for c in range(HD):
            yf_rd[c].wait_recv()
            fold(off_y, c, comm_y, c)
        for c in range(2 * HD, NF):
            yf_rd[c].wait_recv()
            fold(off_y, c, comm_y, c)
        for c in range(HD, NF):
            zf_rd[c].wait_recv()
            fold(off_z, c, comm_z, c)

        for i, (slot, off, c) in enumerate(slots[CQ:]):
            x_rd[CQ + i].wait_recv()
            fold(off, c, comm_x, slot)
        for c in range(NF, CQ):
            x_rd[c].wait_recv()
            out_ref[pl.ds(off_m + c * rows, rows), :] = (
                send_buf[c, :, :] + comm_x[c, :, :]
            )

        for c in range(HD):
            yr_rd[c].wait_recv()
            fold(off_d, c, comm_d, c)
        for c in range(HD):
            zr_rd[c].wait_recv()
            fold(off_d, HD + c, comm_d, HD + c)

        for r in x_rd + yf_rd + zf_rd + yr_rd + zr_rd:
            r.wait_send()

    return pl.pallas_call(
        body,
        out_shape=jax.ShapeDtypeStruct((m, n), jnp.bfloat16),
        in_specs=[pl.BlockSpec(memory_space=pltpu.VMEM)],
        out_specs=pl.BlockSpec(memory_space=pltpu.VMEM),
        scratch_shapes=[
            pltpu.VMEM((NX, rows, n), jnp.bfloat16),
            pltpu.VMEM((NX, rows, n), jnp.bfloat16),
            pltpu.VMEM((NF, rows, n), jnp.bfloat16),
            pltpu.VMEM((NF, rows, n), jnp.bfloat16),
            pltpu.VMEM((ND, rows, n), jnp.bfloat16),
            pltpu.SemaphoreType.DMA((NX,)),
            pltpu.SemaphoreType.DMA((NX,)),
            pltpu.SemaphoreType.DMA((NF,)),
            pltpu.SemaphoreType.DMA((NF,)),
            pltpu.SemaphoreType.DMA((NF,)),
            pltpu.SemaphoreType.DMA((NF,)),
            pltpu.SemaphoreType.DMA((HD,)),
            pltpu.SemaphoreType.DMA((HD,)),
            pltpu.SemaphoreType.DMA((HD,)),
            pltpu.SemaphoreType.DMA((HD,)),
        ],
        compiler_params=pltpu.CompilerParams(collective_id=0),
    )(x)


# device time: 18779 ns/iter; 1.0668x vs baseline; 1.0029x over previous
import jax
import jax.numpy as jnp
from jax import lax
from jax.experimental import pallas as pl
from jax.experimental.pallas import tpu as pltpu

CQ = 16
K2 = 2
K1 = 6
NF = CQ - K2
ND = CQ - K1
HD = ND // 2
NX = CQ + 2 * K2 + K1


def kernel(x):
    m, n = x.shape
    quarter = m // 4
    rows = quarter // CQ

    def body(x_ref, out_ref, send_buf, comm_x, comm_y, comm_z, comm_d,
             x_s, x_r, yf_s, yf_r, zf_s, zf_r, yr_s, yr_r, zr_s, zr_r):
        my_x = lax.axis_index("x")
        my_y = lax.axis_index("y")
        my_z = lax.axis_index("z")
        p = lax.rem(my_z, 2)
        peer_x = (1 - my_x, my_y, my_z)
        peer_y = (my_x, 1 - my_y, my_z)
        peer_z = (my_x, my_y, my_z + 1 - 2 * p)

        off_m = (2 * p + my_y) * quarter
        off_y = (2 * p + (1 - my_y)) * quarter
        off_z = (2 * (1 - p) + my_y) * quarter
        off_d = (2 * (1 - p) + (1 - my_y)) * quarter

        slots = (
            [(c, off_m, c) for c in range(CQ)]
            + [(CQ + i, off_y, NF + i) for i in range(K2)]
            + [(CQ + K2 + i, off_z, NF + i) for i in range(K2)]
            + [(CQ + 2 * K2 + i, off_d, ND + i) for i in range(K1)]
        )

        barrier = pltpu.get_barrier_semaphore()
        for nbr in (peer_x, peer_y, peer_z):
            pl.semaphore_signal(
                barrier, inc=1, device_id=nbr,
                device_id_type=pl.DeviceIdType.MESH,
            )
        pl.semaphore_wait(barrier, 3)

        x_rd = []
        for slot, off, c in slots:
            send_buf[slot, :, :] = (
                x_ref[pl.ds(off + c * rows, rows), :].astype(jnp.bfloat16)
            )
            r = pltpu.make_async_remote_copy(
                src_ref=send_buf.at[slot],
                dst_ref=comm_x.at[slot],
                send_sem=x_s.at[slot],
                recv_sem=x_r.at[slot],
                device_id=peer_x,
                device_id_type=pl.DeviceIdType.MESH,
            )
            r.start()
            x_rd.append(r)

        def fold(off, c, buf, slot):
            out_ref[pl.ds(off + c * rows, rows), :] = (
                x_ref[pl.ds(off + c * rows, rows), :]
                + buf[slot, :, :].astype(jnp.float32)
            ).astype(jnp.bfloat16)

        yf_rd, zf_rd = [], []
        for c in range(NF):
            x_rd[c].wait_recv()
            for dst, ss, rs, peer, lst in (
                (comm_y, yf_s, yf_r, peer_y, yf_rd),
                (comm_z, zf_s, zf_r, peer_z, zf_rd),
            ):
                r = pltpu.make_async_remote_copy(
                    src_ref=comm_x.at[c], dst_ref=dst.at[c],
                    send_sem=ss.at[c], recv_sem=rs.at[c],
                    device_id=peer, device_id_type=pl.DeviceIdType.MESH,
                )
                r.start()
                lst.append(r)
            out_ref[pl.ds(off_m + c * rows, rows), :] = (
                send_buf[c, :, :] + comm_x[c, :, :]
            )

        yr_rd, zr_rd = [], []
        for c in range(HD):
            zf_rd[c].wait_recv()
            r = pltpu.make_async_remote_copy(
                src_ref=comm_z.at[c], dst_ref=comm_d.at[c],
                send_sem=yr_s.at[c], recv_sem=yr_r.at[c],
                device_id=peer_y, device_id_type=pl.DeviceIdType.MESH,
            )
            r.start()
            yr_rd.append(r)
            fold(off_z, c, comm_z, c)
            yf_rd[HD + c].wait_recv()
            r = pltpu.make_async_remote_copy(
                src_ref=comm_y.at[HD + c], dst_ref=comm_d.at[HD + c],
                send_sem=zr_s.at[c], recv_sem=zr_r.at[c],
                device_id=peer_z, device_id_type=pl.DeviceIdType.MESH,
            )
            r.start()
            zr_rd.append(r)
            fold(off_y, HD + c, comm_y, HD + c)

        for c in range(HD):
            yf_rd[c].wait_recv()
            fold(off_y, c, comm_y, c)
        for c in range(2 * HD, NF):
            yf_rd[c].wait_recv()
            fold(off_y, c, comm_y, c)
        for c in range(HD, NF):
            zf_rd[c].wait_recv()
            fold(off_z, c, comm_z, c)

        for i, (slot, off, c) in enumerate(slots[CQ:]):
            x_rd[CQ + i].wait_recv()
            fold(off, c, comm_x, slot)
        for c in range(NF, CQ):
            x_rd[c].wait_recv()
            out_ref[pl.ds(off_m + c * rows, rows), :] = (
                send_buf[c, :, :] + comm_x[c, :, :]
            )

        for c in range(HD):
            yr_rd[c].wait_recv()
            fold(off_d, c, comm_d, c)
        for c in range(HD):
            zr_rd[c].wait_recv()
            fold(off_d, HD + c, comm_d, HD + c)

        for r in x_rd + yf_rd + zf_rd + yr_rd + zr_rd:
            r.wait_send()

    return pl.pallas_call(
        body,
        out_shape=jax.ShapeDtypeStruct((m, n), jnp.bfloat16),
        in_specs=[pl.BlockSpec(memory_space=pltpu.VMEM)],
        out_specs=pl.BlockSpec(memory_space=pltpu.VMEM),
        scratch_shapes=[
            pltpu.VMEM((NX, rows, n), jnp.bfloat16),
            pltpu.VMEM((NX, rows, n), jnp.bfloat16),
            pltpu.VMEM((NF, rows, n), jnp.bfloat16),
            pltpu.VMEM((NF, rows, n), jnp.bfloat16),
            pltpu.VMEM((ND, rows, n), jnp.bfloat16),
            pltpu.SemaphoreType.DMA((NX,)),
            pltpu.SemaphoreType.DMA((NX,)),
            pltpu.SemaphoreType.DMA((NF,)),
            pltpu.SemaphoreType.DMA((NF,)),
            pltpu.SemaphoreType.DMA((NF,)),
            pltpu.SemaphoreType.DMA((NF,)),
            pltpu.SemaphoreType.DMA((HD,)),
            pltpu.SemaphoreType.DMA((HD,)),
            pltpu.SemaphoreType.DMA((HD,)),
            pltpu.SemaphoreType.DMA((HD,)),
        ],
        compiler_params=pltpu.CompilerParams(collective_id=0),
    )(x)
